# baseline (speedup 1.0000x reference)
import functools

import jax
import jax.numpy as jnp
from jax import lax
from jax.experimental import pallas as pl
from jax.experimental.pallas import tpu as pltpu
from jax.experimental.pallas import tpu_sc as plsc

_NC, _NS = 2, 16
_NW = _NC * _NS
_BLK = 128
_CW = 24
_SCALING = 0.2


def _act_cols(x, n):
    col = lax.broadcasted_iota(jnp.int32, x.shape, 1)
    return jnp.where(col < n, jnp.tanh(x), x)


def _dot(a, b):
    return jnp.dot(a, b, preferred_element_type=jnp.float32)


def _mlp(x1, w0, w1, w2, w3):
    x = jnp.maximum(x1 * w0[0:1, :], 0.0)
    x = jnp.maximum(_dot(x, w1), 0.0)
    x = jnp.maximum(_dot(x, w2), 0.0)
    return _dot(x, w3)


def _rowwise(fn, out_widths, arrays, weights, block, rows=None, row_off=0):
    total = arrays[0].shape[0]
    rows = total if rows is None else rows
    nb = rows // block
    assert rows % block == 0
    off_b = row_off // block
    assert row_off % block == 0

    in_specs = [
        pl.BlockSpec((block,) + a.shape[1:],
                     lambda i, _o=off_b, _nd=a.ndim: (i + _o,) + (0,) * (_nd - 1))
        for a in arrays
    ] + [
        pl.BlockSpec(w.shape, lambda i, _nd=w.ndim: (0,) * _nd)
        for w in weights
    ]
    out_specs = [pl.BlockSpec((block, w), lambda i: (i, 0)) for w in out_widths]
    out_shape = [jax.ShapeDtypeStruct((rows, w), jnp.float32) for w in out_widths]
    res = pl.pallas_call(
        fn, grid=(nb,), in_specs=in_specs, out_specs=out_specs,
        out_shape=out_shape)(*arrays, *weights)
    return res


def _k1_body(f_ref, wl1, u1, a1_ref):
    node0 = jnp.tanh(_dot(f_ref[...], wl1[...]))
    a1_ref[...] = _dot(node0, u1[...])


def _ef_body(g_ref, sh_ref, emb_ref, v_ref, f0, f1, f2, f3, out_ref):
    w = _mlp(emb_ref[...], f0[...], f1[...], f2[...], f3[...])
    s = _dot(sh_ref[...], v_ref[...])
    out_ref[...] = g_ref[...] * s * w


def _kmid_body(n_ref, wlin, u, a_ref, *, n_tanh):
    x = _act_cols(_dot(n_ref[...], wlin[...]), n_tanh)
    a_ref[...] = _dot(x, u[...])


def _k5_body(n_ref, wlin3, ub, vb, wc, wh, ws1, ws2,
             nu_ref, nv_ref, nc_ref, nh_ref, scr_ref):
    node = _act_cols(_dot(n_ref[...], wlin3[...]), 16)
    nu_ref[...] = _dot(node, ub[...])
    nv_ref[...] = _dot(node, vb[...])
    nc_ref[...] = _dot(node, wc[...]) * _SCALING
    nh_ref[...] = _dot(node, wh[...]) * _SCALING
    scr_ref[...] = _dot(_act_cols(_dot(node, ws1[...]), 32), ws2[...])


def _kbond_body(x1_ref, x2_ref, ge_ref, b0, b1, b2, b3, wt, wg1, wg2,
                edge_ref, gap_ref):
    wb = _mlp(ge_ref[...], b0[...], b1[...], b2[...], b3[...])
    bf = x1_ref[...] * x2_ref[...] * wb
    edge_ref[...] = _act_cols(bf, 16) @ wt[...] * _SCALING
    gap_ref[...] = _dot(jnp.tanh(_dot(bf, wg1[...])), wg2[...])


_SC_PARAMS = pltpu.CompilerParams(use_tc_tiling_on_sc=False)


def _mesh():
    return plsc.VectorSubcoreMesh(core_axis_name="c", subcore_axis_name="s",
                                  num_cores=_NC, num_subcores=_NS)


def _sc_gather(table, idx):
    n_rows, d = table.shape
    e = idx.shape[0]
    assert e % _BLK == 0
    nblk = e // _BLK

    @functools.partial(
        pl.kernel,
        out_type=jax.ShapeDtypeStruct((e, d), jnp.float32),
        mesh=_mesh(),
        compiler_params=_SC_PARAMS,
        scratch_types=[
            pltpu.VMEM((_BLK,), jnp.int32),
            pltpu.VMEM((_BLK, d), jnp.float32),
            pltpu.SemaphoreType.DMA,
        ],
    )
    def k(table_hbm, idx_hbm, out_hbm, ibuf, rbuf, sem):
        wid = lax.axis_index("s") * _NC + lax.axis_index("c")
        nb_w = (nblk - wid + _NW - 1) // _NW

        def body(j, carry):
            e0 = (wid + j * _NW) * _BLK
            pltpu.sync_copy(idx_hbm.at[pl.ds(e0, _BLK)], ibuf)
            pltpu.async_copy(table_hbm.at[ibuf], rbuf, sem).wait()
            pltpu.sync_copy(rbuf, out_hbm.at[pl.ds(e0, _BLK), :])
            return carry

        lax.fori_loop(0, nb_w, body, 0)

    return k(table, idx)


def _sc_segsum(ef, dst, n_nodes):
    e, d = ef.shape
    assert d % _CW == 0 and e % _BLK == 0 and n_nodes % _NS == 0
    nch = d // _CW
    nblk = e // _BLK
    rpt = n_nodes // _NS
    ef3 = ef.reshape(e, nch, _CW)
    zrows = jnp.zeros((rpt, _CW), jnp.float32)

    @functools.partial(
        pl.kernel,
        out_type=jax.ShapeDtypeStruct((n_nodes, d), jnp.float32),
        mesh=_mesh(),
        compiler_params=_SC_PARAMS,
        scratch_types=[
            pltpu.VMEM((_BLK,), jnp.int32),
            pltpu.VMEM((_BLK, _CW), jnp.float32),
            pltpu.VMEM_SHARED((n_nodes, _CW), jnp.float32),
            pltpu.SemaphoreType.DMA,
        ],
    )
    def k(ef_hbm, dst_hbm, z_hbm, out_hbm, ibuf, rbuf, acc, sem):
        cid = lax.axis_index("c")
        sid = lax.axis_index("s")
        r0 = sid * rpt
        nb_w = (nblk - sid + _NS - 1) // _NS

        for cc in range(_NC):
            for ch in range(cc, nch, _NC):

                @pl.when(cid == cc)
                def _():
                    pltpu.sync_copy(z_hbm, acc.at[pl.ds(r0, rpt), :])
                plsc.subcore_barrier()

                @pl.when(cid == cc)
                def _():
                    def body(j, carry):
                        e0 = (sid + j * _NS) * _BLK
                        pltpu.sync_copy(dst_hbm.at[pl.ds(e0, _BLK)], ibuf)
                        pltpu.sync_copy(ef_hbm.at[pl.ds(e0, _BLK), ch, :], rbuf)
                        pltpu.sync_copy(rbuf, acc.at[ibuf], add=True)
                        return carry

                    lax.fori_loop(0, nb_w, body, 0)
                plsc.subcore_barrier()

                @pl.when(cid == cc)
                def _():
                    pltpu.sync_copy(
                        acc.at[pl.ds(r0, rpt), :],
                        out_hbm.at[pl.ds(r0, rpt), pl.ds(ch * _CW, _CW)])
                plsc.subcore_barrier()

    return k(ef3, dst, zrows)


def _sc_bond_gather(ind, edge_src, edge_dst, emb, node_u, node_v):
    nb = ind.shape[0]
    d = node_u.shape[1]
    nblk = (nb + _BLK - 1) // _BLK
    last0 = nb - _BLK

    @functools.partial(
        pl.kernel,
        out_type=[
            jax.ShapeDtypeStruct((nb, d), jnp.float32),
            jax.ShapeDtypeStruct((nb, d), jnp.float32),
            jax.ShapeDtypeStruct((nb,), jnp.float32),
        ],
        mesh=_mesh(),
        compiler_params=_SC_PARAMS,
        scratch_types=[
            pltpu.VMEM((_BLK,), jnp.int32),
            pltpu.VMEM((_BLK,), jnp.int32),
            pltpu.VMEM((_BLK,), jnp.int32),
            pltpu.VMEM((_BLK,), jnp.float32),
            pltpu.VMEM((_BLK, d), jnp.float32),
            pltpu.VMEM((_BLK, d), jnp.float32),
            pltpu.SemaphoreType.DMA,
            pltpu.SemaphoreType.DMA,
            pltpu.SemaphoreType.DMA,
        ],
    )
    def k(ind_hbm, src_hbm, dst_hbm, emb_hbm, nu_hbm, nv_hbm,
          x1_hbm, x2_hbm, ge_hbm, bbuf, sbuf, dbuf, ebuf, ubuf, vbuf,
          s0, s1, s2):
        wid = lax.axis_index("s") * _NC + lax.axis_index("c")
        nb_w = (nblk - wid + _NW - 1) // _NW

        def body(j, carry):
            b = wid + j * _NW
            e0 = jnp.minimum(b * _BLK, last0)
            pltpu.sync_copy(ind_hbm.at[pl.ds(e0, _BLK)], bbuf)
            d1 = pltpu.async_copy(src_hbm.at[bbuf], sbuf, s0)
            d2 = pltpu.async_copy(dst_hbm.at[bbuf], dbuf, s1)
            d3 = pltpu.async_copy(emb_hbm.at[bbuf], ebuf, s2)
            d1.wait()
            d2.wait()
            d4 = pltpu.async_copy(nu_hbm.at[sbuf], ubuf, s0)
            d5 = pltpu.async_copy(nv_hbm.at[dbuf], vbuf, s1)
            d3.wait()
            pltpu.sync_copy(ebuf, ge_hbm.at[pl.ds(e0, _BLK)])
            d4.wait()
            pltpu.sync_copy(ubuf, x1_hbm.at[pl.ds(e0, _BLK), :])
            d5.wait()
            pltpu.sync_copy(vbuf, x2_hbm.at[pl.ds(e0, _BLK), :])
            return carry

        lax.fori_loop(0, nb_w, body, 0)

    return k(ind, edge_src, edge_dst, emb, node_u, node_v)


def kernel(sh, emb, f_in, edge_src, edge_dst, num_nodes, num_neighbors,
           HH_ind, CC_ind, CH_ind, params):
    p = params
    n = f_in.shape[0]
    e = sh.shape[0]
    nb = HH_ind.shape[0]
    inv = 1.0 / jnp.sqrt(jnp.asarray(num_neighbors).astype(jnp.float32))
    inv = inv + (jnp.asarray(num_nodes) * 0).astype(jnp.float32)
    wl2s = p['W_lin2'] * inv
    wl3s = p['W_lin3'] * inv

    def _blk(rows, target):
        b = min(rows, target) & ~7
        while rows % b:
            b -= 8
        return b

    blk_n = _blk(n, 2000)
    blk_e = _blk(e, 3200)
    blk_b = _blk(nb, 2000)

    (a1,) = _rowwise(_k1_body, [72], [f_in], [p['W_lin1'], p['U1']], blk_n)
    g1 = _sc_gather(a1, edge_src)
    (ef1,) = _rowwise(_ef_body, [72], [g1, sh, emb],
                      [p['V1']] + list(p['fc1']), blk_e)
    n1 = _sc_segsum(ef1, edge_dst, n)

    (a2,) = _rowwise(functools.partial(_kmid_body, n_tanh=8), [144],
                     [n1], [wl2s, p['U2']], blk_n)
    g2 = _sc_gather(a2, edge_src)
    (ef2,) = _rowwise(_ef_body, [144], [g2, sh, emb],
                      [p['V2']] + list(p['fc2']), blk_e)
    n2 = _sc_segsum(ef2, edge_dst, n)

    node_u, node_v, node_c, node_h, screen = _rowwise(
        _k5_body, [144, 144, 196, 25, 6],
        [n2], [wl3s, p['Ub'], p['Vb'], p['W_C'], p['W_H'],
               p['W_s1'], p['W_s2']], blk_n)

    ind_all = jnp.concatenate([HH_ind, CC_ind, CH_ind])
    x1, x2, ge = _sc_bond_gather(ind_all, edge_src, edge_dst, emb[:, 0],
                                 node_u, node_v)
    ge = ge[:, None]

    bond_ws = list(p['fcb'])
    outs = {}
    for t, (name, wt) in enumerate(
            [('HH', p['W_HH']), ('CC', p['W_CC']), ('CH', p['W_CH'])]):
        edge_t, gap_t = _rowwise(
            _kbond_body, [wt.shape[1], 3], [x1, x2, ge],
            bond_ws + [wt, p['W_g1'], p['W_g2']], blk_b,
            rows=nb, row_off=t * nb)
        outs[name] = (edge_t, gap_t)

    return (node_h, node_c, outs['HH'][0], outs['CH'][0], outs['CC'][0],
            screen, outs['CC'][1], outs['HH'][1], outs['CH'][1])
